# baseline (speedup 1.0000x reference)
import jax
import jax.numpy as jnp
from jax.experimental import pallas as pl

H = 32
W = 32
D = 256


def _body(col_ref, row_ref, out_ref):
    col = col_ref[...]
    row = row_ref[...]
    col_t = col.T
    row_t = row.T
    top = jnp.broadcast_to(col_t[:, None, :], (D, H, W))
    bot = jnp.broadcast_to(row_t[:, :, None], (D, H, W))
    out_ref[0, :D] = top
    out_ref[0, D:] = bot


def kernel(x, row_embed, col_embed):
    b = x.shape[0]
    out = pl.pallas_call(
        _body,
        grid=(b,),
        in_specs=[
            pl.BlockSpec((W, D), lambda i: (0, 0)),
            pl.BlockSpec((H, D), lambda i: (0, 0)),
        ],
        out_specs=pl.BlockSpec((1, 2 * D, H, W), lambda i: (i, 0, 0, 0)),
        out_shape=jax.ShapeDtypeStruct((b, 2 * D, H, W), jnp.float32),
    )(col_embed[:W], row_embed[:H])
    return out


# speedup vs baseline: 6.5387x; 6.5387x over previous
import jax
import jax.numpy as jnp
from jax.experimental import pallas as pl

H = 32
W = 32
D = 256


def _body(col_ref, row_ref, out_ref):
    col = col_ref[...]
    for h in range(H):
        out_ref[0, h, :, :D] = col
        out_ref[0, h, :, D:] = jnp.broadcast_to(row_ref[h, :][None, :], (W, D))


def kernel(x, row_embed, col_embed):
    b = x.shape[0]
    out = pl.pallas_call(
        _body,
        grid=(b,),
        in_specs=[
            pl.BlockSpec((W, D), lambda i: (0, 0)),
            pl.BlockSpec((H, D), lambda i: (0, 0)),
        ],
        out_specs=pl.BlockSpec((1, H, W, 2 * D), lambda i: (i, 0, 0, 0)),
        out_shape=jax.ShapeDtypeStruct((b, H, W, 2 * D), jnp.float32),
    )(col_embed[:W], row_embed[:H])
    return jnp.transpose(out, (0, 3, 1, 2))


# speedup vs baseline: 8.9157x; 1.3635x over previous
import jax
import jax.numpy as jnp
from jax.experimental import pallas as pl

H = 32
W = 32
D = 256


def _body(col_ref, row_ref, out_ref):
    col = col_ref[...]
    for h in range(H):
        out_ref[0, h, :, :D] = col
        out_ref[0, h, :, D:] = jnp.broadcast_to(row_ref[h, :][None, :], (W, D))


def kernel(x, row_embed, col_embed):
    b = x.shape[0]
    out = pl.pallas_call(
        _body,
        grid=(b,),
        in_specs=[
            pl.BlockSpec((W, D), lambda i: (0, 0)),
            pl.BlockSpec((H, D), lambda i: (0, 0)),
        ],
        out_specs=pl.BlockSpec((1, H, W, 2 * D), lambda i: (i, 0, 0, 0)),
        out_shape=jax.ShapeDtypeStruct((b, H, W, 2 * D), jnp.float32),
    )(col_embed, row_embed)
    return jnp.transpose(out, (0, 3, 1, 2))
